# baseline (speedup 1.0000x reference)
import functools

import jax
import jax.numpy as jnp
from jax import lax
from jax.experimental import pallas as pl
from jax.experimental.pallas import tpu as pltpu
from jax.experimental.pallas import tpu_sc as plsc

B = 16384
F = 26
V = 100000
D = 32
ND = 13
OUT = 128
HID = F * D

NC = 2
NS = 16
NW = NC * NS
PH = HID // 2
UPW = PH // NW
BH = B // 2
VH0 = 49920
VH1 = V - VH0


def _sc_gather_t(tbl_fd, idx_t, u_base):
    mesh = plsc.VectorSubcoreMesh(core_axis_name="c", subcore_axis_name="s",
                                  num_cores=NC, num_subcores=NS)

    @functools.partial(
        pl.kernel,
        out_type=jax.ShapeDtypeStruct((PH, B), jnp.float32),
        mesh=mesh,
        scratch_types=[
            pltpu.VMEM((VH0,), jnp.float32),
            pltpu.VMEM((VH1,), jnp.float32),
            pltpu.VMEM((B,), jnp.int32),
            pltpu.VMEM((BH,), jnp.float32),
            pltpu.SemaphoreType.DMA,
            pltpu.SemaphoreType.DMA,
        ],
        compiler_params=pltpu.CompilerParams(needs_layout_passes=False),
    )
    def gather_kernel(tbl_hbm, idx_hbm, out_hbm, stage_a, stage_b, idx_v,
                      out_v, sem_a, sem_b):
        wid = lax.axis_index("s") * NC + lax.axis_index("c")
        o0 = wid * UPW
        u0 = u_base + o0

        def _pass(stage, h, bh, first):
            lo = 0 if h == 0 else VH0

            @plsc.parallel_loop(0, BH // 16, 1, unroll=8)
            def body(i):
                vi = idx_v[pl.ds(bh * BH + i * 16, 16)]
                if h == 0:
                    m = vi < VH0
                else:
                    m = vi >= VH0
                vl = jnp.where(m, vi - lo, 0)
                vals = plsc.load_gather(stage, [vl], mask=m)
                vals = jnp.where(m, vals, 0.0)
                if first:
                    out_v[pl.ds(i * 16, 16)] = vals
                else:
                    plsc.addupdate(out_v.at[pl.ds(i * 16, 16)], vals)

        pltpu.async_copy(tbl_hbm.at[u0, pl.ds(0, VH0)], stage_a, sem_a)
        pltpu.async_copy(tbl_hbm.at[u0, pl.ds(VH0, VH1)], stage_b, sem_b)

        def unit(k, _):
            uu = u0 + k

            @pl.when(jnp.logical_or(k == 0, (uu % D) == 0))
            def _():
                pltpu.sync_copy(idx_hbm.at[uu // D], idx_v)

            pltpu.make_async_copy(tbl_hbm.at[uu, pl.ds(0, VH0)],
                                  stage_a, sem_a).wait()
            _pass(stage_a, 0, 0, True)
            pltpu.make_async_copy(tbl_hbm.at[uu, pl.ds(VH0, VH1)],
                                  stage_b, sem_b).wait()
            _pass(stage_b, 1, 0, False)
            pltpu.sync_copy(out_v, out_hbm.at[o0 + k, pl.ds(0, BH)])
            _pass(stage_a, 0, 1, True)

            @pl.when(k < UPW - 1)
            def _():
                pltpu.async_copy(tbl_hbm.at[uu + 1, pl.ds(0, VH0)],
                                 stage_a, sem_a)

            _pass(stage_b, 1, 1, False)
            pltpu.sync_copy(out_v, out_hbm.at[o0 + k, pl.ds(BH, BH)])

            @pl.when(k < UPW - 1)
            def _():
                pltpu.async_copy(tbl_hbm.at[uu + 1, pl.ds(VH0, VH1)],
                                 stage_b, sem_b)

            return 0

        lax.fori_loop(0, UPW, unit, 0)

    return gather_kernel(tbl_fd, idx_t)


def _mlp_part_body(g_ref, w1_ref, out_ref):
    out_ref[...] = lax.dot_general(g_ref[...], w1_ref[...],
                                   (((0,), (0,)), ((), ())),
                                   preferred_element_type=jnp.float32)


def _tc_part(g, w1a):
    bs = 2048
    return pl.pallas_call(
        _mlp_part_body,
        grid=(B // bs,),
        in_specs=[
            pl.BlockSpec((PH, bs), lambda i: (0, i)),
            pl.BlockSpec((PH, OUT), lambda i: (0, 0)),
        ],
        out_specs=pl.BlockSpec((bs, OUT), lambda i: (i, 0)),
        out_shape=jax.ShapeDtypeStruct((B, OUT), jnp.float32),
        compiler_params=pltpu.CompilerParams(
            dimension_semantics=("arbitrary",),
        ),
    )(g, w1a)


def _mlp_fin_body(g_ref, acc_ref, dense_ref, w1_ref, w2_ref, b_ref, out_ref):
    acc = lax.dot_general(g_ref[...], w1_ref[...],
                          (((0,), (0,)), ((), ())),
                          preferred_element_type=jnp.float32)
    acc = acc + acc_ref[...]
    acc = acc + jnp.dot(dense_ref[...], w2_ref[...],
                        preferred_element_type=jnp.float32)
    acc = acc + b_ref[...]
    out_ref[...] = jnp.maximum(acc, 0.0)


def _tc_fin(g, part, dense_p, w1b, w2_p, b2d):
    bs = 2048
    return pl.pallas_call(
        _mlp_fin_body,
        grid=(B // bs,),
        in_specs=[
            pl.BlockSpec((PH, bs), lambda i: (0, i)),
            pl.BlockSpec((bs, OUT), lambda i: (i, 0)),
            pl.BlockSpec((bs, 16), lambda i: (i, 0)),
            pl.BlockSpec((PH, OUT), lambda i: (0, 0)),
            pl.BlockSpec((16, OUT), lambda i: (0, 0)),
            pl.BlockSpec((1, OUT), lambda i: (0, 0)),
        ],
        out_specs=pl.BlockSpec((bs, OUT), lambda i: (i, 0)),
        out_shape=jax.ShapeDtypeStruct((B, OUT), jnp.float32),
        compiler_params=pltpu.CompilerParams(
            dimension_semantics=("arbitrary",),
        ),
    )(g, part, dense_p, w1b, w2_p, b2d)


def kernel(sparse_indices, dense_features, tables, W, b):
    tbl_fd = jnp.transpose(tables, (0, 2, 1)).reshape(HID, V)
    idx_t = jnp.transpose(sparse_indices, (1, 0))
    g1 = _sc_gather_t(tbl_fd, idx_t, 0)
    g2 = _sc_gather_t(tbl_fd, idx_t, PH)
    dense_p = jnp.pad(dense_features, ((0, 0), (0, 16 - ND)))
    w2_p = jnp.pad(W[HID:], ((0, 16 - ND), (0, 0)))
    part = _tc_part(g1, W[:PH])
    return _tc_fin(g2, part, dense_p, W[PH:HID], w2_p, b.reshape(1, OUT))


# speedup vs baseline: 1.0545x; 1.0545x over previous
import functools

import jax
import jax.numpy as jnp
from jax import lax
from jax.experimental import pallas as pl
from jax.experimental.pallas import tpu as pltpu
from jax.experimental.pallas import tpu_sc as plsc

B = 16384
F = 26
V = 100000
D = 32
ND = 13
OUT = 128
HID = F * D

NC = 2
NS = 16
NW = NC * NS
UPW = HID // NW
BH = B // 2
VH0 = 49920
VH1 = V - VH0


def _sc_gather_t(tbl_fd, idx_t):
    mesh = plsc.VectorSubcoreMesh(core_axis_name="c", subcore_axis_name="s",
                                  num_cores=NC, num_subcores=NS)

    @functools.partial(
        pl.kernel,
        out_type=jax.ShapeDtypeStruct((HID, B), jnp.float32),
        mesh=mesh,
        scratch_types=[
            pltpu.VMEM((VH0,), jnp.float32),
            pltpu.VMEM((VH1,), jnp.float32),
            pltpu.VMEM((B,), jnp.int32),
            pltpu.VMEM((BH,), jnp.float32),
            pltpu.SemaphoreType.DMA,
            pltpu.SemaphoreType.DMA,
        ],
        compiler_params=pltpu.CompilerParams(needs_layout_passes=False),
    )
    def gather_kernel(tbl_hbm, idx_hbm, out_hbm, stage_a, stage_b, idx_v,
                      out_v, sem_a, sem_b):
        wid = lax.axis_index("s") * NC + lax.axis_index("c")
        u0 = wid * UPW

        def _pass(stage, h, bh, first):
            lo = 0 if h == 0 else VH0

            @plsc.parallel_loop(0, BH // 16, 1, unroll=16)
            def body(i):
                vi = idx_v[pl.ds(bh * BH + i * 16, 16)]
                if h == 0:
                    m = vi < VH0
                    vl = vi
                else:
                    m = vi >= VH0
                    vl = vi - lo
                vals = plsc.load_gather(stage, [vl], mask=m)
                vals = jnp.where(m, vals, 0.0)
                if first:
                    out_v[pl.ds(i * 16, 16)] = vals
                else:
                    plsc.addupdate(out_v.at[pl.ds(i * 16, 16)], vals)

        pltpu.async_copy(tbl_hbm.at[u0, pl.ds(0, VH0)], stage_a, sem_a)
        pltpu.async_copy(tbl_hbm.at[u0, pl.ds(VH0, VH1)], stage_b, sem_b)

        def unit(k, _):
            uu = u0 + k

            @pl.when(jnp.logical_or(k == 0, (uu % D) == 0))
            def _():
                pltpu.sync_copy(idx_hbm.at[uu // D], idx_v)

            pltpu.make_async_copy(tbl_hbm.at[uu, pl.ds(0, VH0)],
                                  stage_a, sem_a).wait()
            _pass(stage_a, 0, 0, True)
            pltpu.make_async_copy(tbl_hbm.at[uu, pl.ds(VH0, VH1)],
                                  stage_b, sem_b).wait()
            _pass(stage_b, 1, 0, False)
            pltpu.sync_copy(out_v, out_hbm.at[uu, pl.ds(0, BH)])
            _pass(stage_a, 0, 1, True)

            @pl.when(k < UPW - 1)
            def _():
                pltpu.async_copy(tbl_hbm.at[uu + 1, pl.ds(0, VH0)],
                                 stage_a, sem_a)

            _pass(stage_b, 1, 1, False)
            pltpu.sync_copy(out_v, out_hbm.at[uu, pl.ds(BH, BH)])

            @pl.when(k < UPW - 1)
            def _():
                pltpu.async_copy(tbl_hbm.at[uu + 1, pl.ds(VH0, VH1)],
                                 stage_b, sem_b)

            return 0

        lax.fori_loop(0, UPW, unit, 0)

    return gather_kernel(tbl_fd, idx_t)


def _mlp_body(g_ref, dense_ref, w1_ref, w2_ref, b_ref, out_ref):
    acc = lax.dot_general(g_ref[...], w1_ref[...],
                          (((0,), (0,)), ((), ())),
                          preferred_element_type=jnp.float32)
    acc = acc + jnp.dot(dense_ref[...], w2_ref[...],
                        preferred_element_type=jnp.float32)
    acc = acc + b_ref[...]
    out_ref[...] = jnp.maximum(acc, 0.0)


def _tc_mlp(g, dense_p, w1, w2_p, b2d):
    bs = 2048
    grid = (B // bs,)
    return pl.pallas_call(
        _mlp_body,
        grid=grid,
        in_specs=[
            pl.BlockSpec((HID, bs), lambda i: (0, i)),
            pl.BlockSpec((bs, 16), lambda i: (i, 0)),
            pl.BlockSpec((HID, OUT), lambda i: (0, 0)),
            pl.BlockSpec((16, OUT), lambda i: (0, 0)),
            pl.BlockSpec((1, OUT), lambda i: (0, 0)),
        ],
        out_specs=pl.BlockSpec((bs, OUT), lambda i: (i, 0)),
        out_shape=jax.ShapeDtypeStruct((B, OUT), jnp.float32),
        compiler_params=pltpu.CompilerParams(
            dimension_semantics=("arbitrary",),
        ),
    )(g, dense_p, w1, w2_p, b2d)


def kernel(sparse_indices, dense_features, tables, W, b):
    tbl_fd = jnp.transpose(tables, (0, 2, 1)).reshape(HID, V)
    idx_t = jnp.transpose(sparse_indices, (1, 0))
    g = _sc_gather_t(tbl_fd, idx_t)
    dense_p = jnp.pad(dense_features, ((0, 0), (0, 16 - ND)))
    w1 = W[:HID]
    w2_p = jnp.pad(W[HID:], ((0, 16 - ND), (0, 0)))
    return _tc_mlp(g, dense_p, w1, w2_p, b.reshape(1, OUT))
